# Initial kernel scaffold; baseline (speedup 1.0000x reference)
import jax
import jax.numpy as jnp
from jax.experimental import pallas as pl


def kernel(x, edge_index, W1, b1, W2, b2, Wp1, bp1, Wp2, bp2):
    raise NotImplementedError("write your pallas kernel here")


# speedup vs baseline: 8.0310x; 8.0310x over previous
import functools

import jax
import jax.numpy as jnp
from jax import lax
from jax.experimental import pallas as pl
from jax.experimental.pallas import tpu as pltpu
from jax.experimental.pallas import tpu_sc as plsc

N = 10000
E = 320000
NACC = 10240
NC = 2
NS = 16
NW = NC * NS
B = 128
CHUNKS = 80
EPAD = NW * CHUNKS * B
RPT = NACC // NS
BLK = 1024

_mesh = plsc.VectorSubcoreMesh(core_axis_name="c", subcore_axis_name="s")


def _deg_body(dst_hbm, ones_hbm, out_hbm, idx_v, ones_v, buf_v, acc_sh, sem):
    c = lax.axis_index("c")
    s = lax.axis_index("s")
    w = s * NC + c
    pltpu.sync_copy(ones_hbm.at[0], ones_v)
    pltpu.sync_copy(ones_hbm.at[1], buf_v)
    base = s * RPT
    for k in range(RPT // B):
        pltpu.sync_copy(buf_v, acc_sh.at[pl.ds(base + k * B, B)])
    pltpu.sync_copy(dst_hbm.at[w], idx_v)
    plsc.subcore_barrier()

    def step(j, carry):
        pltpu.sync_copy(ones_v, acc_sh.at[idx_v.at[j]], add=True)
        return carry

    lax.fori_loop(0, CHUNKS, step, 0)
    plsc.subcore_barrier()
    out_base = c * NACC + s * RPT
    for k in range(RPT // B):
        pltpu.sync_copy(acc_sh.at[pl.ds(base + k * B, B)], buf_v)
        pltpu.sync_copy(buf_v, out_hbm.at[pl.ds(out_base + k * B, B)])


_deg_kernel = pl.kernel(
    _deg_body,
    out_type=jax.ShapeDtypeStruct((2 * NACC, 16), jnp.float32),
    mesh=_mesh,
    scratch_types=[
        pltpu.VMEM((CHUNKS, B), jnp.int32),
        pltpu.VMEM((B, 16), jnp.float32),
        pltpu.VMEM((B, 16), jnp.float32),
        pltpu.VMEM_SHARED((NACC, 16), jnp.float32),
        pltpu.SemaphoreType.DMA,
    ],
)


def _make_scatter(D):
    def body(hs_hbm, src_hbm, dst_hbm, zero_hbm, out_hbm,
             sidx_v, didx_v, rows_v, acc_sh, sem):
        c = lax.axis_index("c")
        s = lax.axis_index("s")
        w = s * NC + c
        pltpu.sync_copy(zero_hbm, rows_v)
        base = s * RPT
        for k in range(RPT // B):
            pltpu.sync_copy(rows_v, acc_sh.at[pl.ds(base + k * B, B)])
        pltpu.sync_copy(src_hbm.at[w], sidx_v)
        pltpu.sync_copy(dst_hbm.at[w], didx_v)
        plsc.subcore_barrier()

        def step(j, carry):
            pltpu.async_copy(hs_hbm.at[sidx_v.at[j]], rows_v, sem).wait()
            pltpu.sync_copy(rows_v, acc_sh.at[didx_v.at[j]], add=True)
            return carry

        lax.fori_loop(0, CHUNKS, step, 0)
        plsc.subcore_barrier()
        out_base = c * NACC + s * RPT
        for k in range(RPT // B):
            pltpu.sync_copy(acc_sh.at[pl.ds(base + k * B, B)], rows_v)
            pltpu.sync_copy(rows_v, out_hbm.at[pl.ds(out_base + k * B, B)])

    return pl.kernel(
        body,
        out_type=jax.ShapeDtypeStruct((2 * NACC, D), jnp.float32),
        mesh=_mesh,
        scratch_types=[
            pltpu.VMEM((CHUNKS, B), jnp.int32),
            pltpu.VMEM((CHUNKS, B), jnp.int32),
            pltpu.VMEM((B, D), jnp.float32),
            pltpu.VMEM_SHARED((NACC, D), jnp.float32),
            pltpu.SemaphoreType.DMA,
        ],
    )


_scatter128 = _make_scatter(128)


def _tc1_body(dega, degb, x, W1, dis_o, hs1_o):
    deg = 1.0 + dega[:, 0:1] + degb[:, 0:1]
    dis = lax.rsqrt(deg)
    dis_o[...] = dis
    hs1_o[...] = jnp.dot(dis * x[...], W1[...],
                         preferred_element_type=jnp.float32)


def _tc2_body(a, b, hs1, dis_r, b1, W2, hs2_o):
    dis = dis_r[...]
    t = jnp.maximum(dis * (a[...] + b[...] + hs1[...]) + b1[...], 0.0)
    h2 = jnp.dot(dis * t, W2[...], preferred_element_type=jnp.float32)
    hs2_o[...] = jnp.concatenate(
        [h2, jnp.zeros((h2.shape[0], 128 - h2.shape[1]), jnp.float32)],
        axis=1)


def _tc3_body(a, b, hs2, dis_r, b2, Wp1, bp1, Wp2, bp2, out_o):
    dis = dis_r[...]
    o2 = jnp.maximum(
        dis * (a[...][:, :64] + b[...][:, :64] + hs2[...][:, :64])
        + b2[...], 0.0)
    p = jnp.maximum(
        jnp.dot(o2, Wp1[...], preferred_element_type=jnp.float32) + bp1[...],
        0.0)
    out_o[...] = jnp.dot(p, Wp2[...], preferred_element_type=jnp.float32) \
        + bp2[...]


def _row(shape):
    return pl.BlockSpec(shape, lambda i: (i, 0))


def _full(shape):
    return pl.BlockSpec(shape, lambda i: (0, 0))


_GRID = NACC // BLK

_tc1 = pl.pallas_call(
    _tc1_body,
    grid=(_GRID,),
    in_specs=[
        _row((BLK, 16)),
        pl.BlockSpec((BLK, 16), lambda i: (i + _GRID, 0)),
        _row((BLK, 128)),
        _full((128, 128)),
    ],
    out_specs=[_row((BLK, 1)), _row((BLK, 128))],
    out_shape=[
        jax.ShapeDtypeStruct((NACC, 1), jnp.float32),
        jax.ShapeDtypeStruct((NACC, 128), jnp.float32),
    ],
)

_tc2 = pl.pallas_call(
    _tc2_body,
    grid=(_GRID,),
    in_specs=[
        _row((BLK, 128)),
        pl.BlockSpec((BLK, 128), lambda i: (i + _GRID, 0)),
        _row((BLK, 128)),
        _row((BLK, 1)),
        _full((1, 128)),
        _full((128, 64)),
    ],
    out_specs=_row((BLK, 128)),
    out_shape=jax.ShapeDtypeStruct((NACC, 128), jnp.float32),
)

_tc3 = pl.pallas_call(
    _tc3_body,
    grid=(_GRID,),
    in_specs=[
        _row((BLK, 128)),
        pl.BlockSpec((BLK, 128), lambda i: (i + _GRID, 0)),
        _row((BLK, 128)),
        _row((BLK, 1)),
        _full((1, 64)),
        _full((64, 32)),
        _full((1, 32)),
        _full((32, 16)),
        _full((1, 16)),
    ],
    out_specs=_row((BLK, 16)),
    out_shape=jax.ShapeDtypeStruct((NACC, 16), jnp.float32),
)


def kernel(x, edge_index, W1, b1, W2, b2, Wp1, bp1, Wp2, bp2):
    src = edge_index[0].astype(jnp.int32)
    dst = edge_index[1].astype(jnp.int32)
    npad = EPAD - E
    pad_dst = N + (jnp.arange(npad, dtype=jnp.int32) % (NACC - N))
    src3 = jnp.concatenate([src, jnp.zeros((npad,), jnp.int32)])
    src3 = src3.reshape(NW, CHUNKS, B)
    dst3 = jnp.concatenate([dst, pad_dst]).reshape(NW, CHUNKS, B)

    ones16 = jnp.stack([jnp.ones((B, 16), jnp.float32),
                        jnp.zeros((B, 16), jnp.float32)])
    z128 = jnp.zeros((B, 128), jnp.float32)

    xp = jnp.concatenate(
        [x, jnp.zeros((NACC - N, 128), jnp.float32)], axis=0)

    degp = _deg_kernel(dst3, ones16)
    dis, hs1 = _tc1(degp, degp, xp, W1)
    a1 = _scatter128(hs1, src3, dst3, z128)
    hs2 = _tc2(a1, a1, hs1, dis, b1.reshape(1, -1), W2)
    a2 = _scatter128(hs2, src3, dst3, z128)
    out = _tc3(a2, a2, hs2, dis, b2.reshape(1, -1), Wp1,
               bp1.reshape(1, -1), Wp2, bp2.reshape(1, -1))
    return out[:N]


# speedup vs baseline: 26.4737x; 3.2964x over previous
import functools

import jax
import jax.numpy as jnp
from jax import lax
from jax.experimental import pallas as pl
from jax.experimental.pallas import tpu as pltpu
from jax.experimental.pallas import tpu_sc as plsc

N = 10000
E = 320000
NACC = 10240
NC = 2
NS = 16
NW = NC * NS
B = 128
CHUNKS = 80
EPAD = NW * CHUNKS * B
RPT = NACC // NS
SB = 2
SBC = CHUNKS // SB
BLK = 1024

_mesh = plsc.VectorSubcoreMesh(core_axis_name="c", subcore_axis_name="s")


def _deg_body(dst_hbm, ones_hbm, out_hbm, idx_v, ones_v, buf_v, acc_sh, sem):
    c = lax.axis_index("c")
    s = lax.axis_index("s")
    w = s * NC + c
    pltpu.sync_copy(ones_hbm.at[0], ones_v)
    pltpu.sync_copy(ones_hbm.at[1], buf_v)
    base = s * RPT
    for k in range(RPT // B):
        pltpu.sync_copy(buf_v, acc_sh.at[pl.ds(base + k * B, B)])
    pltpu.sync_copy(dst_hbm.at[w], idx_v)
    plsc.subcore_barrier()

    def step(j, carry):
        pltpu.sync_copy(ones_v, acc_sh.at[idx_v.at[j]], add=True)
        return carry

    lax.fori_loop(0, CHUNKS, step, 0)
    plsc.subcore_barrier()
    out_base = c * NACC + s * RPT
    for k in range(RPT // B):
        pltpu.sync_copy(acc_sh.at[pl.ds(base + k * B, B)], buf_v)
        pltpu.sync_copy(buf_v, out_hbm.at[pl.ds(out_base + k * B, B)])


_deg_kernel = pl.kernel(
    _deg_body,
    out_type=jax.ShapeDtypeStruct((2 * NACC, 16), jnp.float32),
    mesh=_mesh,
    scratch_types=[
        pltpu.VMEM((CHUNKS, B), jnp.int32),
        pltpu.VMEM((B, 16), jnp.float32),
        pltpu.VMEM((B, 16), jnp.float32),
        pltpu.VMEM_SHARED((NACC, 16), jnp.float32),
        pltpu.SemaphoreType.DMA,
    ],
)


def _make_scatter(D):
    def body(hs_hbm, src_hbm, dst_hbm, zero_hbm, out_hbm,
             sidx_v, didx_v, rows_v, acc_sh, gsem, ssem):
        c = lax.axis_index("c")
        s = lax.axis_index("s")
        w = s * NC + c
        pltpu.sync_copy(zero_hbm, rows_v.at[0])
        base = s * RPT
        for k in range(RPT // B):
            pltpu.sync_copy(rows_v.at[0], acc_sh.at[pl.ds(base + k * B, B)])
        plsc.subcore_barrier()

        for sb in range(SB):
            pltpu.sync_copy(src_hbm.at[w].at[pl.ds(sb * SBC, SBC)], sidx_v)
            pltpu.sync_copy(dst_hbm.at[w].at[pl.ds(sb * SBC, SBC)], didx_v)

            def pair(p, carry):
                j0 = 2 * p
                g0 = pltpu.async_copy(
                    hs_hbm.at[sidx_v.at[j0]], rows_v.at[0], gsem)
                g1 = pltpu.async_copy(
                    hs_hbm.at[sidx_v.at[j0 + 1]], rows_v.at[1], gsem)
                g0.wait()
                s0 = pltpu.async_copy(
                    rows_v.at[0], acc_sh.at[didx_v.at[j0]], ssem, add=True)
                g1.wait()
                s1 = pltpu.async_copy(
                    rows_v.at[1], acc_sh.at[didx_v.at[j0 + 1]], ssem,
                    add=True)
                s0.wait()
                s1.wait()
                return carry

            lax.fori_loop(0, SBC // 2, pair, 0)
        plsc.subcore_barrier()
        out_base = c * NACC + s * RPT
        for k in range(RPT // B):
            pltpu.sync_copy(acc_sh.at[pl.ds(base + k * B, B)], rows_v.at[0])
            pltpu.sync_copy(rows_v.at[0],
                            out_hbm.at[pl.ds(out_base + k * B, B)])

    return pl.kernel(
        body,
        out_type=jax.ShapeDtypeStruct((2 * NACC, D), jnp.float32),
        mesh=_mesh,
        scratch_types=[
            pltpu.VMEM((SBC, B), jnp.int32),
            pltpu.VMEM((SBC, B), jnp.int32),
            pltpu.VMEM((2, B, D), jnp.float32),
            pltpu.VMEM_SHARED((NACC, D), jnp.float32),
            pltpu.SemaphoreType.DMA,
            pltpu.SemaphoreType.DMA,
        ],
    )


_scatter128 = _make_scatter(128)


def _tc1_body(dega, degb, x, W1, dis_o, hs1_o):
    deg = 1.0 + dega[:, 0:1] + degb[:, 0:1]
    dis = lax.rsqrt(deg)
    dis_o[...] = dis
    hs1_o[...] = jnp.dot(dis * x[...], W1[...],
                         preferred_element_type=jnp.float32)


def _tc2_body(a, b, hs1, dis_r, b1, W2, hs2_o):
    dis = dis_r[...]
    t = jnp.maximum(dis * (a[...] + b[...] + hs1[...]) + b1[...], 0.0)
    h2 = jnp.dot(dis * t, W2[...], preferred_element_type=jnp.float32)
    hs2_o[...] = jnp.concatenate(
        [h2, jnp.zeros((h2.shape[0], 128 - h2.shape[1]), jnp.float32)],
        axis=1)


def _tc3_body(a, b, hs2, dis_r, b2, Wp1, bp1, Wp2, bp2, out_o):
    dis = dis_r[...]
    o2 = jnp.maximum(
        dis * (a[...][:, :64] + b[...][:, :64] + hs2[...][:, :64])
        + b2[...], 0.0)
    p = jnp.maximum(
        jnp.dot(o2, Wp1[...], preferred_element_type=jnp.float32) + bp1[...],
        0.0)
    out_o[...] = jnp.dot(p, Wp2[...], preferred_element_type=jnp.float32) \
        + bp2[...]


def _row(shape):
    return pl.BlockSpec(shape, lambda i: (i, 0))


def _full(shape):
    return pl.BlockSpec(shape, lambda i: (0, 0))


_GRID = NACC // BLK

_tc1 = pl.pallas_call(
    _tc1_body,
    grid=(_GRID,),
    in_specs=[
        _row((BLK, 16)),
        pl.BlockSpec((BLK, 16), lambda i: (i + _GRID, 0)),
        _row((BLK, 128)),
        _full((128, 128)),
    ],
    out_specs=[_row((BLK, 1)), _row((BLK, 128))],
    out_shape=[
        jax.ShapeDtypeStruct((NACC, 1), jnp.float32),
        jax.ShapeDtypeStruct((NACC, 128), jnp.float32),
    ],
)

_tc2 = pl.pallas_call(
    _tc2_body,
    grid=(_GRID,),
    in_specs=[
        _row((BLK, 128)),
        pl.BlockSpec((BLK, 128), lambda i: (i + _GRID, 0)),
        _row((BLK, 128)),
        _row((BLK, 1)),
        _full((1, 128)),
        _full((128, 64)),
    ],
    out_specs=_row((BLK, 128)),
    out_shape=jax.ShapeDtypeStruct((NACC, 128), jnp.float32),
)

_tc3 = pl.pallas_call(
    _tc3_body,
    grid=(_GRID,),
    in_specs=[
        _row((BLK, 128)),
        pl.BlockSpec((BLK, 128), lambda i: (i + _GRID, 0)),
        _row((BLK, 128)),
        _row((BLK, 1)),
        _full((1, 64)),
        _full((64, 32)),
        _full((1, 32)),
        _full((32, 16)),
        _full((1, 16)),
    ],
    out_specs=_row((BLK, 16)),
    out_shape=jax.ShapeDtypeStruct((NACC, 16), jnp.float32),
)


def kernel(x, edge_index, W1, b1, W2, b2, Wp1, bp1, Wp2, bp2):
    src = edge_index[0].astype(jnp.int32)
    dst = edge_index[1].astype(jnp.int32)
    npad = EPAD - E
    pad_dst = N + (jnp.arange(npad, dtype=jnp.int32) % (NACC - N))
    src3 = jnp.concatenate([src, pad_dst])
    src3 = src3.reshape(NW, CHUNKS, B)
    dst3 = jnp.concatenate([dst, pad_dst]).reshape(NW, CHUNKS, B)

    ones16 = jnp.stack([jnp.ones((B, 16), jnp.float32),
                        jnp.zeros((B, 16), jnp.float32)])
    z128 = jnp.zeros((B, 128), jnp.float32)

    xp = jnp.concatenate(
        [x, jnp.zeros((NACC - N, 128), jnp.float32)], axis=0)

    degp = _deg_kernel(dst3, ones16)
    dis, hs1 = _tc1(degp, degp, xp, W1)
    a1 = _scatter128(hs1, src3, dst3, z128)
    hs2 = _tc2(a1, a1, hs1, dis, b1.reshape(1, -1), W2)
    a2 = _scatter128(hs2, src3, dst3, z128)
    out = _tc3(a2, a2, hs2, dis, b2.reshape(1, -1), Wp1,
               bp1.reshape(1, -1), Wp2, bp2.reshape(1, -1))
    return out[:N]


# speedup vs baseline: 29.2678x; 1.1055x over previous
import functools

import jax
import jax.numpy as jnp
from jax import lax
from jax.experimental import pallas as pl
from jax.experimental.pallas import tpu as pltpu
from jax.experimental.pallas import tpu_sc as plsc

N = 10000
E = 320000
NACC = 10240
NC = 2
NS = 16
NW = NC * NS
B = 128
CHUNKS = 80
EPAD = NW * CHUNKS * B
RPT = NACC // NS
SB = 2
SBC = CHUNKS // SB
BLK = 1024

_mesh = plsc.VectorSubcoreMesh(core_axis_name="c", subcore_axis_name="s")


def _deg_body(dst_hbm, ones_hbm, out_hbm, idx_v, ones_v, buf_v, acc_sh, sem):
    c = lax.axis_index("c")
    s = lax.axis_index("s")
    w = s * NC + c
    pltpu.sync_copy(ones_hbm.at[0], ones_v)
    pltpu.sync_copy(ones_hbm.at[1], buf_v)
    base = s * RPT
    for k in range(RPT // B):
        pltpu.sync_copy(buf_v, acc_sh.at[pl.ds(base + k * B, B)])
    pltpu.sync_copy(dst_hbm.at[w], idx_v)
    plsc.subcore_barrier()

    def step(j, carry):
        pltpu.sync_copy(ones_v, acc_sh.at[idx_v.at[j]], add=True)
        return carry

    lax.fori_loop(0, CHUNKS, step, 0)
    plsc.subcore_barrier()
    out_base = c * NACC + s * RPT
    for k in range(RPT // B):
        pltpu.sync_copy(acc_sh.at[pl.ds(base + k * B, B)], buf_v)
        pltpu.sync_copy(buf_v, out_hbm.at[pl.ds(out_base + k * B, B)])


_deg_kernel = pl.kernel(
    _deg_body,
    out_type=jax.ShapeDtypeStruct((2 * NACC, 16), jnp.float32),
    mesh=_mesh,
    scratch_types=[
        pltpu.VMEM((CHUNKS, B), jnp.int32),
        pltpu.VMEM((B, 16), jnp.float32),
        pltpu.VMEM((B, 16), jnp.float32),
        pltpu.VMEM_SHARED((NACC, 16), jnp.float32),
        pltpu.SemaphoreType.DMA,
    ],
)


def _make_scatter(D, tc_tiling=True):
    def body(hs_hbm, src_hbm, dst_hbm, zero_hbm, out_hbm,
             sidx_v, didx_v, rows_v, acc_sh, gsem, ssem):
        c = lax.axis_index("c")
        s = lax.axis_index("s")
        w = s * NC + c
        pltpu.sync_copy(zero_hbm, rows_v.at[0])
        base = s * RPT
        for k in range(RPT // B):
            pltpu.sync_copy(rows_v.at[0], acc_sh.at[pl.ds(base + k * B, B)])
        plsc.subcore_barrier()

        for sb in range(SB):
            pltpu.sync_copy(src_hbm.at[w].at[pl.ds(sb * SBC, SBC)], sidx_v)
            pltpu.sync_copy(dst_hbm.at[w].at[pl.ds(sb * SBC, SBC)], didx_v)

            def pair(p, carry):
                j0 = 2 * p
                g0 = pltpu.async_copy(
                    hs_hbm.at[sidx_v.at[j0]], rows_v.at[0], gsem)
                g1 = pltpu.async_copy(
                    hs_hbm.at[sidx_v.at[j0 + 1]], rows_v.at[1], gsem)
                g0.wait()
                s0 = pltpu.async_copy(
                    rows_v.at[0], acc_sh.at[didx_v.at[j0]], ssem, add=True)
                g1.wait()
                s1 = pltpu.async_copy(
                    rows_v.at[1], acc_sh.at[didx_v.at[j0 + 1]], ssem,
                    add=True)
                s0.wait()
                s1.wait()
                return carry

            lax.fori_loop(0, SBC // 2, pair, 0)
        plsc.subcore_barrier()
        out_base = c * NACC + s * RPT
        for k in range(RPT // B):
            pltpu.sync_copy(acc_sh.at[pl.ds(base + k * B, B)], rows_v.at[0])
            pltpu.sync_copy(rows_v.at[0],
                            out_hbm.at[pl.ds(out_base + k * B, B)])

    params = None
    if not tc_tiling:
        params = pltpu.CompilerParams(use_tc_tiling_on_sc=False)
    return pl.kernel(
        body,
        out_type=jax.ShapeDtypeStruct((2 * NACC, D), jnp.float32),
        mesh=_mesh,
        compiler_params=params,
        scratch_types=[
            pltpu.VMEM((SBC, B), jnp.int32),
            pltpu.VMEM((SBC, B), jnp.int32),
            pltpu.VMEM((2, B, D), jnp.float32),
            pltpu.VMEM_SHARED((NACC, D), jnp.float32),
            pltpu.SemaphoreType.DMA,
            pltpu.SemaphoreType.DMA,
        ],
    )


_scatter128 = _make_scatter(128)
_scatter64 = _make_scatter(64, tc_tiling=False)


def _tc1_body(dega, degb, x, W1, dis_o, hs1_o):
    deg = 1.0 + dega[:, 0:1] + degb[:, 0:1]
    dis = lax.rsqrt(deg)
    dis_o[...] = dis
    hs1_o[...] = jnp.dot(dis * x[...], W1[...],
                         preferred_element_type=jnp.float32)


def _tc2_body(a, b, hs1, dis_r, b1, W2, hs2_o):
    dis = dis_r[...]
    t = jnp.maximum(dis * (a[...] + b[...] + hs1[...]) + b1[...], 0.0)
    hs2_o[...] = jnp.dot(dis * t, W2[...], preferred_element_type=jnp.float32)


def _tc3_body(a, b, hs2, dis_r, b2, Wp1, bp1, Wp2, bp2, out_o):
    dis = dis_r[...]
    o2 = jnp.maximum(dis * (a[...] + b[...] + hs2[...]) + b2[...], 0.0)
    p = jnp.maximum(
        jnp.dot(o2, Wp1[...], preferred_element_type=jnp.float32) + bp1[...],
        0.0)
    out_o[...] = jnp.dot(p, Wp2[...], preferred_element_type=jnp.float32) \
        + bp2[...]


def _row(shape):
    return pl.BlockSpec(shape, lambda i: (i, 0))


def _full(shape):
    return pl.BlockSpec(shape, lambda i: (0, 0))


_GRID = NACC // BLK

_tc1 = pl.pallas_call(
    _tc1_body,
    grid=(_GRID,),
    in_specs=[
        _row((BLK, 16)),
        pl.BlockSpec((BLK, 16), lambda i: (i + _GRID, 0)),
        _row((BLK, 128)),
        _full((128, 128)),
    ],
    out_specs=[_row((BLK, 1)), _row((BLK, 128))],
    out_shape=[
        jax.ShapeDtypeStruct((NACC, 1), jnp.float32),
        jax.ShapeDtypeStruct((NACC, 128), jnp.float32),
    ],
)

_tc2 = pl.pallas_call(
    _tc2_body,
    grid=(_GRID,),
    in_specs=[
        _row((BLK, 128)),
        pl.BlockSpec((BLK, 128), lambda i: (i + _GRID, 0)),
        _row((BLK, 128)),
        _row((BLK, 1)),
        _full((1, 128)),
        _full((128, 64)),
    ],
    out_specs=_row((BLK, 64)),
    out_shape=jax.ShapeDtypeStruct((NACC, 64), jnp.float32),
)

_tc3 = pl.pallas_call(
    _tc3_body,
    grid=(_GRID,),
    in_specs=[
        _row((BLK, 64)),
        pl.BlockSpec((BLK, 64), lambda i: (i + _GRID, 0)),
        _row((BLK, 64)),
        _row((BLK, 1)),
        _full((1, 64)),
        _full((64, 32)),
        _full((1, 32)),
        _full((32, 16)),
        _full((1, 16)),
    ],
    out_specs=_row((BLK, 16)),
    out_shape=jax.ShapeDtypeStruct((NACC, 16), jnp.float32),
)


def kernel(x, edge_index, W1, b1, W2, b2, Wp1, bp1, Wp2, bp2):
    src = edge_index[0].astype(jnp.int32)
    dst = edge_index[1].astype(jnp.int32)
    npad = EPAD - E
    pad_dst = N + (jnp.arange(npad, dtype=jnp.int32) % (NACC - N))
    src3 = jnp.concatenate([src, pad_dst])
    src3 = src3.reshape(NW, CHUNKS, B)
    dst3 = jnp.concatenate([dst, pad_dst]).reshape(NW, CHUNKS, B)

    ones16 = jnp.stack([jnp.ones((B, 16), jnp.float32),
                        jnp.zeros((B, 16), jnp.float32)])
    z128 = jnp.zeros((B, 128), jnp.float32)
    z64 = jnp.zeros((B, 64), jnp.float32)

    xp = jnp.concatenate(
        [x, jnp.zeros((NACC - N, 128), jnp.float32)], axis=0)

    degp = _deg_kernel(dst3, ones16)
    dis, hs1 = _tc1(degp, degp, xp, W1)
    a1 = _scatter128(hs1, src3, dst3, z128)
    hs2 = _tc2(a1, a1, hs1, dis, b1.reshape(1, -1), W2)
    a2 = _scatter64(hs2, src3, dst3, z64)
    out = _tc3(a2, a2, hs2, dis, b2.reshape(1, -1), Wp1,
               bp1.reshape(1, -1), Wp2, bp2.reshape(1, -1))
    return out[:N]


# speedup vs baseline: 30.1145x; 1.0289x over previous
import jax
import jax.numpy as jnp
from jax import lax
from jax.experimental import pallas as pl
from jax.experimental.pallas import tpu as pltpu
from jax.experimental.pallas import tpu_sc as plsc

N = 10000
E = 320000
NACC = 10240
NC = 2
NS = 16
NW = NC * NS
B = 128
CHUNKS = 80
EPAD = NW * CHUNKS * B
RPT = NACC // NS
SB = 2
SBC = CHUNKS // SB
NBUF = 2
BLK = 1024

_mesh = plsc.VectorSubcoreMesh(core_axis_name="c", subcore_axis_name="s")


def _deg_body(dst_hbm, ones_hbm, out_hbm, idx_v, ones_v, buf_v, acc_sh,
              gsem, ssem):
    c = lax.axis_index("c")
    s = lax.axis_index("s")
    w = s * NC + c
    pltpu.sync_copy(ones_hbm.at[0], ones_v)
    pltpu.sync_copy(ones_hbm.at[1], buf_v)
    base = s * RPT
    zw = []
    for k in range(RPT // B):
        zw.append(pltpu.async_copy(
            buf_v, acc_sh.at[pl.ds(base + k * B, B)], ssem))
    for d in zw:
        d.wait()
    pltpu.sync_copy(dst_hbm.at[w], idx_v)
    plsc.subcore_barrier()

    def step(p, carry):
        ds = [pltpu.async_copy(
            ones_v, acc_sh.at[idx_v.at[8 * p + i]], ssem, add=True)
            for i in range(8)]
        for d in ds:
            d.wait()
        return carry

    lax.fori_loop(0, CHUNKS // 8, step, 0)
    plsc.subcore_barrier()
    out_base = c * NACC + s * RPT
    rd = {}
    wr = {}
    nck = RPT // B
    for k in range(nck):
        b = k % 2
        if k >= 2:
            wr[k - 2].wait()
        rd[k] = pltpu.async_copy(
            acc_sh.at[pl.ds(base + k * B, B)],
            buf_v if b == 0 else ones_v, gsem)
        rd[k].wait()
        wr[k] = pltpu.async_copy(
            buf_v if b == 0 else ones_v,
            out_hbm.at[pl.ds(out_base + k * B, B)], ssem)
    wr[nck - 2].wait()
    wr[nck - 1].wait()


_deg_kernel = pl.kernel(
    _deg_body,
    out_type=jax.ShapeDtypeStruct((2 * NACC, 16), jnp.float32),
    mesh=_mesh,
    scratch_types=[
        pltpu.VMEM((CHUNKS, B), jnp.int32),
        pltpu.VMEM((B, 16), jnp.float32),
        pltpu.VMEM((B, 16), jnp.float32),
        pltpu.VMEM_SHARED((NACC, 16), jnp.float32),
        pltpu.SemaphoreType.DMA,
        pltpu.SemaphoreType.DMA,
    ],
)


def _make_scatter(D, tc_tiling=True):
    def body(hs_hbm, src_hbm, dst_hbm, zero_hbm, out_hbm,
             sidx_v, didx_v, rows_v, acc_sh, gsem, ssem):
        c = lax.axis_index("c")
        s = lax.axis_index("s")
        w = s * NC + c
        pltpu.sync_copy(zero_hbm, rows_v.at[0])
        base = s * RPT
        zw = []
        for k in range(RPT // B):
            zw.append(pltpu.async_copy(
                rows_v.at[0], acc_sh.at[pl.ds(base + k * B, B)], ssem))
        for d in zw:
            d.wait()
        plsc.subcore_barrier()

        for sb in range(SB):
            pltpu.sync_copy(src_hbm.at[w].at[pl.ds(sb * SBC, SBC)], sidx_v)
            pltpu.sync_copy(dst_hbm.at[w].at[pl.ds(sb * SBC, SBC)], didx_v)

            def group(p, carry):
                j0 = NBUF * p
                gs = [pltpu.async_copy(
                    hs_hbm.at[sidx_v.at[j0 + i]], rows_v.at[i], gsem)
                    for i in range(NBUF)]
                ss = []
                for i in range(NBUF):
                    gs[i].wait()
                    ss.append(pltpu.async_copy(
                        rows_v.at[i], acc_sh.at[didx_v.at[j0 + i]], ssem,
                        add=True))
                for d in ss:
                    d.wait()
                return carry

            lax.fori_loop(0, SBC // NBUF, group, 0)
        plsc.subcore_barrier()

        out_base = c * NACC + s * RPT
        rd = {}
        wr = {}
        nck = RPT // B
        for k in range(nck):
            b = k % NBUF
            if k >= NBUF:
                wr[k - NBUF].wait()
            rd[k] = pltpu.async_copy(
                acc_sh.at[pl.ds(base + k * B, B)], rows_v.at[b], gsem)
            rd[k].wait()
            wr[k] = pltpu.async_copy(
                rows_v.at[b], out_hbm.at[pl.ds(out_base + k * B, B)], ssem)
        for k in range(max(nck - NBUF, 0), nck):
            wr[k].wait()

    params = None
    if not tc_tiling:
        params = pltpu.CompilerParams(use_tc_tiling_on_sc=False)
    return pl.kernel(
        body,
        out_type=jax.ShapeDtypeStruct((2 * NACC, D), jnp.float32),
        mesh=_mesh,
        compiler_params=params,
        scratch_types=[
            pltpu.VMEM((SBC, B), jnp.int32),
            pltpu.VMEM((SBC, B), jnp.int32),
            pltpu.VMEM((NBUF, B, D), jnp.float32),
            pltpu.VMEM_SHARED((NACC, D), jnp.float32),
            pltpu.SemaphoreType.DMA,
            pltpu.SemaphoreType.DMA,
        ],
    )


_scatter128 = _make_scatter(128)
_scatter64 = _make_scatter(64, tc_tiling=False)


def _tc1_body(dega, degb, x, W1, dis_o, hs1_o):
    deg = 1.0 + dega[:, 0:1] + degb[:, 0:1]
    dis = lax.rsqrt(deg)
    dis_o[...] = dis
    hs1_o[...] = jnp.dot(dis * x[...], W1[...],
                         preferred_element_type=jnp.float32)


def _tc2_body(a, b, hs1, dis_r, b1, W2, hs2_o):
    dis = dis_r[...]
    t = jnp.maximum(dis * (a[...] + b[...] + hs1[...]) + b1[...], 0.0)
    hs2_o[...] = jnp.dot(dis * t, W2[...], preferred_element_type=jnp.float32)


def _tc3_body(a, b, hs2, dis_r, b2, Wp1, bp1, Wp2, bp2, out_o):
    dis = dis_r[...]
    o2 = jnp.maximum(dis * (a[...] + b[...] + hs2[...]) + b2[...], 0.0)
    p = jnp.maximum(
        jnp.dot(o2, Wp1[...], preferred_element_type=jnp.float32) + bp1[...],
        0.0)
    out_o[...] = jnp.dot(p, Wp2[...], preferred_element_type=jnp.float32) \
        + bp2[...]


def _row(shape):
    return pl.BlockSpec(shape, lambda i: (i, 0))


def _full(shape):
    return pl.BlockSpec(shape, lambda i: (0, 0))


_GRID = NACC // BLK

_tc1 = pl.pallas_call(
    _tc1_body,
    grid=(_GRID,),
    in_specs=[
        _row((BLK, 16)),
        pl.BlockSpec((BLK, 16), lambda i: (i + _GRID, 0)),
        _row((BLK, 128)),
        _full((128, 128)),
    ],
    out_specs=[_row((BLK, 1)), _row((BLK, 128))],
    out_shape=[
        jax.ShapeDtypeStruct((NACC, 1), jnp.float32),
        jax.ShapeDtypeStruct((NACC, 128), jnp.float32),
    ],
)

_tc2 = pl.pallas_call(
    _tc2_body,
    grid=(_GRID,),
    in_specs=[
        _row((BLK, 128)),
        pl.BlockSpec((BLK, 128), lambda i: (i + _GRID, 0)),
        _row((BLK, 128)),
        _row((BLK, 1)),
        _full((1, 128)),
        _full((128, 64)),
    ],
    out_specs=_row((BLK, 64)),
    out_shape=jax.ShapeDtypeStruct((NACC, 64), jnp.float32),
)

_tc3 = pl.pallas_call(
    _tc3_body,
    grid=(_GRID,),
    in_specs=[
        _row((BLK, 64)),
        pl.BlockSpec((BLK, 64), lambda i: (i + _GRID, 0)),
        _row((BLK, 64)),
        _row((BLK, 1)),
        _full((1, 64)),
        _full((64, 32)),
        _full((1, 32)),
        _full((32, 16)),
        _full((1, 16)),
    ],
    out_specs=_row((BLK, 16)),
    out_shape=jax.ShapeDtypeStruct((NACC, 16), jnp.float32),
)


def kernel(x, edge_index, W1, b1, W2, b2, Wp1, bp1, Wp2, bp2):
    src = edge_index[0].astype(jnp.int32)
    dst = edge_index[1].astype(jnp.int32)
    npad = EPAD - E
    pad_dst = N + (jnp.arange(npad, dtype=jnp.int32) % (NACC - N))
    src3 = jnp.concatenate([src, pad_dst])
    src3 = src3.reshape(NW, CHUNKS, B)
    dst3 = jnp.concatenate([dst, pad_dst]).reshape(NW, CHUNKS, B)

    ones16 = jnp.stack([jnp.ones((B, 16), jnp.float32),
                        jnp.zeros((B, 16), jnp.float32)])
    z128 = jnp.zeros((B, 128), jnp.float32)
    z64 = jnp.zeros((B, 64), jnp.float32)

    xp = jnp.concatenate(
        [x, jnp.zeros((NACC - N, 128), jnp.float32)], axis=0)

    degp = _deg_kernel(dst3, ones16)
    dis, hs1 = _tc1(degp, degp, xp, W1)
    a1 = _scatter128(hs1, src3, dst3, z128)
    hs2 = _tc2(a1, a1, hs1, dis, b1.reshape(1, -1), W2)
    a2 = _scatter64(hs2, src3, dst3, z64)
    out = _tc3(a2, a2, hs2, dis, b2.reshape(1, -1), Wp1,
               bp1.reshape(1, -1), Wp2, bp2.reshape(1, -1))
    return out[:N]


# speedup vs baseline: 30.3351x; 1.0073x over previous
import jax
import jax.numpy as jnp
from jax import lax
from jax.experimental import pallas as pl
from jax.experimental.pallas import tpu as pltpu
from jax.experimental.pallas import tpu_sc as plsc

N = 10000
E = 320000
NACC = 10240
NC = 2
NS = 16
NW = NC * NS
B = 128
CHUNKS = 80
EPAD = NW * CHUNKS * B
RPT = NACC // NS
SB = 2
SBC = CHUNKS // SB
NBUF = 2
BLK = 1000

_mesh = plsc.VectorSubcoreMesh(core_axis_name="c", subcore_axis_name="s")


def _deg_body(dst_hbm, ones_hbm, out_hbm, idx_v, ones_v, buf_v, acc_sh,
              gsem, ssem):
    c = lax.axis_index("c")
    s = lax.axis_index("s")
    w = s * NC + c
    pltpu.sync_copy(ones_hbm.at[0], ones_v)
    pltpu.sync_copy(ones_hbm.at[1], buf_v)
    base = s * RPT
    zw = []
    for k in range(RPT // B):
        zw.append(pltpu.async_copy(
            buf_v, acc_sh.at[pl.ds(base + k * B, B)], ssem))
    for d in zw:
        d.wait()
    pltpu.sync_copy(dst_hbm.at[w], idx_v)
    plsc.subcore_barrier()

    def step(p, carry):
        ds = [pltpu.async_copy(
            ones_v, acc_sh.at[idx_v.at[8 * p + i]], ssem, add=True)
            for i in range(8)]
        for d in ds:
            d.wait()
        return carry

    lax.fori_loop(0, CHUNKS // 8, step, 0)
    plsc.subcore_barrier()
    out_pl = out_hbm.at[c]
    rd = {}
    wr = {}
    nck = RPT // B
    for k in range(nck):
        b = k % 2
        if k >= 2:
            wr[k - 2].wait()
        rd[k] = pltpu.async_copy(
            acc_sh.at[pl.ds(base + k * B, B)],
            buf_v if b == 0 else ones_v, gsem)
        rd[k].wait()
        wr[k] = pltpu.async_copy(
            buf_v if b == 0 else ones_v,
            out_pl.at[pl.ds(base + k * B, B)], ssem)
    wr[nck - 2].wait()
    wr[nck - 1].wait()


_deg_kernel = pl.kernel(
    _deg_body,
    out_type=jax.ShapeDtypeStruct((2, NACC, 16), jnp.float32),
    mesh=_mesh,
    scratch_types=[
        pltpu.VMEM((CHUNKS, B), jnp.int32),
        pltpu.VMEM((B, 16), jnp.float32),
        pltpu.VMEM((B, 16), jnp.float32),
        pltpu.VMEM_SHARED((NACC, 16), jnp.float32),
        pltpu.SemaphoreType.DMA,
        pltpu.SemaphoreType.DMA,
    ],
)


def _make_scatter(D, tc_tiling=True):
    def body(hs_hbm, src_hbm, dst_hbm, zero_hbm, out_hbm,
             sidx_v, didx_v, rows_v, acc_sh, gsem, ssem):
        c = lax.axis_index("c")
        s = lax.axis_index("s")
        w = s * NC + c
        pltpu.sync_copy(zero_hbm, rows_v.at[0])
        base = s * RPT
        zw = []
        for k in range(RPT // B):
            zw.append(pltpu.async_copy(
                rows_v.at[0], acc_sh.at[pl.ds(base + k * B, B)], ssem))
        for d in zw:
            d.wait()
        plsc.subcore_barrier()

        for sb in range(SB):
            pltpu.sync_copy(src_hbm.at[w].at[pl.ds(sb * SBC, SBC)], sidx_v)
            pltpu.sync_copy(dst_hbm.at[w].at[pl.ds(sb * SBC, SBC)], didx_v)

            def group(p, carry):
                j0 = NBUF * p
                gs = [pltpu.async_copy(
                    hs_hbm.at[sidx_v.at[j0 + i]], rows_v.at[i], gsem)
                    for i in range(NBUF)]
                ss = []
                for i in range(NBUF):
                    gs[i].wait()
                    ss.append(pltpu.async_copy(
                        rows_v.at[i], acc_sh.at[didx_v.at[j0 + i]], ssem,
                        add=True))
                for d in ss:
                    d.wait()
                return carry

            lax.fori_loop(0, SBC // NBUF, group, 0)
        plsc.subcore_barrier()

        out_pl = out_hbm.at[c]
        rd = {}
        wr = {}
        nck = RPT // B
        for k in range(nck):
            b = k % NBUF
            if k >= NBUF:
                wr[k - NBUF].wait()
            rd[k] = pltpu.async_copy(
                acc_sh.at[pl.ds(base + k * B, B)], rows_v.at[b], gsem)
            rd[k].wait()
            wr[k] = pltpu.async_copy(
                rows_v.at[b], out_pl.at[pl.ds(base + k * B, B)], ssem)
        for k in range(max(nck - NBUF, 0), nck):
            wr[k].wait()

    params = None
    if not tc_tiling:
        params = pltpu.CompilerParams(use_tc_tiling_on_sc=False)
    return pl.kernel(
        body,
        out_type=jax.ShapeDtypeStruct((2, NACC, D), jnp.float32),
        mesh=_mesh,
        compiler_params=params,
        scratch_types=[
            pltpu.VMEM((SBC, B), jnp.int32),
            pltpu.VMEM((SBC, B), jnp.int32),
            pltpu.VMEM((NBUF, B, D), jnp.float32),
            pltpu.VMEM_SHARED((NACC, D), jnp.float32),
            pltpu.SemaphoreType.DMA,
            pltpu.SemaphoreType.DMA,
        ],
    )


_scatter128 = _make_scatter(128)
_scatter64 = _make_scatter(64, tc_tiling=False)


def _tc1_body(dega, degb, x, W1, dis_o, hs1_o):
    deg = 1.0 + dega[0, :, 0:1] + degb[0, :, 0:1]
    dis = lax.rsqrt(deg)
    dis_o[...] = dis
    hs1_o[...] = jnp.dot(dis * x[...], W1[...],
                         preferred_element_type=jnp.float32)


def _tc2_body(a, b, hs1, dis_r, b1, W2, hs2_o):
    dis = dis_r[...]
    t = jnp.maximum(dis * (a[0] + b[0] + hs1[...]) + b1[...], 0.0)
    hs2_o[...] = jnp.dot(dis * t, W2[...], preferred_element_type=jnp.float32)


def _tc3_body(a, b, hs2, dis_r, b2, Wp1, bp1, Wp2, bp2, out_o):
    dis = dis_r[...]
    o2 = jnp.maximum(dis * (a[0] + b[0] + hs2[...]) + b2[...], 0.0)
    p = jnp.maximum(
        jnp.dot(o2, Wp1[...], preferred_element_type=jnp.float32) + bp1[...],
        0.0)
    out_o[...] = jnp.dot(p, Wp2[...], preferred_element_type=jnp.float32) \
        + bp2[...]


def _row(shape):
    return pl.BlockSpec(shape, lambda i: (i, 0))


def _full(shape):
    return pl.BlockSpec(shape, lambda i: (0, 0))


_GRID = N // BLK


def _plane(d, p):
    return pl.BlockSpec((1, BLK, d), lambda i, _p=p: (_p, i, 0))


_tc1 = pl.pallas_call(
    _tc1_body,
    grid=(_GRID,),
    in_specs=[
        _plane(16, 0),
        _plane(16, 1),
        _row((BLK, 128)),
        _full((128, 128)),
    ],
    out_specs=[_row((BLK, 1)), _row((BLK, 128))],
    out_shape=[
        jax.ShapeDtypeStruct((N, 1), jnp.float32),
        jax.ShapeDtypeStruct((N, 128), jnp.float32),
    ],
)

_tc2 = pl.pallas_call(
    _tc2_body,
    grid=(_GRID,),
    in_specs=[
        _plane(128, 0),
        _plane(128, 1),
        _row((BLK, 128)),
        _row((BLK, 1)),
        _full((1, 128)),
        _full((128, 64)),
    ],
    out_specs=_row((BLK, 64)),
    out_shape=jax.ShapeDtypeStruct((N, 64), jnp.float32),
)

_tc3 = pl.pallas_call(
    _tc3_body,
    grid=(_GRID,),
    in_specs=[
        _plane(64, 0),
        _plane(64, 1),
        _row((BLK, 64)),
        _row((BLK, 1)),
        _full((1, 64)),
        _full((64, 32)),
        _full((1, 32)),
        _full((32, 16)),
        _full((1, 16)),
    ],
    out_specs=_row((BLK, 16)),
    out_shape=jax.ShapeDtypeStruct((N, 16), jnp.float32),
)


def kernel(x, edge_index, W1, b1, W2, b2, Wp1, bp1, Wp2, bp2):
    src = edge_index[0].astype(jnp.int32)
    dst = edge_index[1].astype(jnp.int32)
    npad = EPAD - E
    idx = jnp.arange(npad, dtype=jnp.int32)
    pad_dst = N + idx % (NACC - N)
    pad_src = idx % N
    src3 = jnp.concatenate([src, pad_src]).reshape(NW, CHUNKS, B)
    dst3 = jnp.concatenate([dst, pad_dst]).reshape(NW, CHUNKS, B)

    ones16 = jnp.stack([jnp.ones((B, 16), jnp.float32),
                        jnp.zeros((B, 16), jnp.float32)])
    z128 = jnp.zeros((B, 128), jnp.float32)
    z64 = jnp.zeros((B, 64), jnp.float32)

    degp = _deg_kernel(dst3, ones16)
    dis, hs1 = _tc1(degp, degp, x, W1)
    a1 = _scatter128(hs1, src3, dst3, z128)
    hs2 = _tc2(a1, a1, hs1, dis, b1.reshape(1, -1), W2)
    a2 = _scatter64(hs2, src3, dst3, z64)
    return _tc3(a2, a2, hs2, dis, b2.reshape(1, -1), Wp1,
                bp1.reshape(1, -1), Wp2, bp2.reshape(1, -1))


# speedup vs baseline: 30.4812x; 1.0048x over previous
import jax
import jax.numpy as jnp
from jax import lax
from jax.experimental import pallas as pl
from jax.experimental.pallas import tpu as pltpu
from jax.experimental.pallas import tpu_sc as plsc

N = 10000
E = 320000
NACC = 10240
NC = 2
NS = 16
NW = NC * NS
B = 125
CHUNKS = 80
RPT = NACC // NS
SB = 2
SBC = CHUNKS // SB
NBUF = 2
CS = 80
BLK = 1000

_mesh = plsc.VectorSubcoreMesh(core_axis_name="c", subcore_axis_name="s")


def _deg_body(dst_hbm, ones_hbm, out_hbm, idx_v, ones_v, buf_v, acc_sh,
              gsem, ssem):
    c = lax.axis_index("c")
    s = lax.axis_index("s")
    w = s * NC + c
    pltpu.sync_copy(ones_hbm.at[0], ones_v)
    pltpu.sync_copy(ones_hbm.at[1], buf_v)
    base = s * RPT
    zw = []
    for k in range(RPT // CS):
        zw.append(pltpu.async_copy(
            buf_v.at[pl.ds(0, CS)], acc_sh.at[pl.ds(base + k * CS, CS)],
            ssem))
    for d in zw:
        d.wait()
    pltpu.sync_copy(dst_hbm.at[w], idx_v)
    plsc.subcore_barrier()

    def step(p, carry):
        ds = [pltpu.async_copy(
            ones_v, acc_sh.at[idx_v.at[8 * p + i]], ssem, add=True)
            for i in range(8)]
        for d in ds:
            d.wait()
        return carry

    lax.fori_loop(0, CHUNKS // 8, step, 0)
    plsc.subcore_barrier()
    out_pl = out_hbm.at[c]
    rd = {}
    wr = {}
    nck = RPT // CS
    for k in range(nck):
        bv = buf_v if k % 2 == 0 else ones_v
        if k >= 2:
            wr[k - 2].wait()
        rd[k] = pltpu.async_copy(
            acc_sh.at[pl.ds(base + k * CS, CS)], bv.at[pl.ds(0, CS)], gsem)
        rd[k].wait()
        wr[k] = pltpu.async_copy(
            bv.at[pl.ds(0, CS)], out_pl.at[pl.ds(base + k * CS, CS)], ssem)
    wr[nck - 2].wait()
    wr[nck - 1].wait()


_deg_kernel = pl.kernel(
    _deg_body,
    out_type=jax.ShapeDtypeStruct((2, NACC, 16), jnp.float32),
    mesh=_mesh,
    scratch_types=[
        pltpu.VMEM((CHUNKS, B), jnp.int32),
        pltpu.VMEM((B, 16), jnp.float32),
        pltpu.VMEM((B, 16), jnp.float32),
        pltpu.VMEM_SHARED((NACC, 16), jnp.float32),
        pltpu.SemaphoreType.DMA,
        pltpu.SemaphoreType.DMA,
    ],
)


def _make_scatter(D, tc_tiling=True):
    def body(hs_hbm, src_hbm, dst_hbm, zero_hbm, out_hbm,
             sidx_v, didx_v, rows_v, acc_sh, gsem, ssem):
        c = lax.axis_index("c")
        s = lax.axis_index("s")
        w = s * NC + c
        pltpu.sync_copy(zero_hbm, rows_v.at[0].at[pl.ds(0, CS)])
        base = s * RPT
        zw = []
        for k in range(RPT // CS):
            zw.append(pltpu.async_copy(
                rows_v.at[0].at[pl.ds(0, CS)],
                acc_sh.at[pl.ds(base + k * CS, CS)], ssem))
        for d in zw:
            d.wait()
        plsc.subcore_barrier()

        for sb in range(SB):
            pltpu.sync_copy(src_hbm.at[w].at[pl.ds(sb * SBC, SBC)], sidx_v)
            pltpu.sync_copy(dst_hbm.at[w].at[pl.ds(sb * SBC, SBC)], didx_v)

            def group(p, carry):
                j0 = NBUF * p
                gs = [pltpu.async_copy(
                    hs_hbm.at[sidx_v.at[j0 + i]], rows_v.at[i], gsem)
                    for i in range(NBUF)]
                ss = []
                for i in range(NBUF):
                    gs[i].wait()
                    ss.append(pltpu.async_copy(
                        rows_v.at[i], acc_sh.at[didx_v.at[j0 + i]], ssem,
                        add=True))
                for d in ss:
                    d.wait()
                return carry

            lax.fori_loop(0, SBC // NBUF, group, 0)
        plsc.subcore_barrier()

        out_pl = out_hbm.at[c]
        rd = {}
        wr = {}
        nck = RPT // CS
        for k in range(nck):
            bv = rows_v.at[k % NBUF].at[pl.ds(0, CS)]
            if k >= NBUF:
                wr[k - NBUF].wait()
            rd[k] = pltpu.async_copy(
                acc_sh.at[pl.ds(base + k * CS, CS)], bv, gsem)
            rd[k].wait()
            wr[k] = pltpu.async_copy(
                bv, out_pl.at[pl.ds(base + k * CS, CS)], ssem)
        for k in range(max(nck - NBUF, 0), nck):
            wr[k].wait()

    params = None
    if not tc_tiling:
        params = pltpu.CompilerParams(use_tc_tiling_on_sc=False)
    return pl.kernel(
        body,
        out_type=jax.ShapeDtypeStruct((2, NACC, D), jnp.float32),
        mesh=_mesh,
        compiler_params=params,
        scratch_types=[
            pltpu.VMEM((SBC, B), jnp.int32),
            pltpu.VMEM((SBC, B), jnp.int32),
            pltpu.VMEM((NBUF, B, D), jnp.float32),
            pltpu.VMEM_SHARED((NACC, D), jnp.float32),
            pltpu.SemaphoreType.DMA,
            pltpu.SemaphoreType.DMA,
        ],
    )


_scatter128 = _make_scatter(128)
_scatter64 = _make_scatter(64, tc_tiling=False)


def _tc1_body(dega, degb, x, W1, dis_o, hs1_o):
    deg = 1.0 + dega[0, :, 0:1] + degb[0, :, 0:1]
    dis = lax.rsqrt(deg)
    dis_o[...] = dis
    hs1_o[...] = jnp.dot(dis * x[...], W1[...],
                         preferred_element_type=jnp.float32)


def _tc2_body(a, b, hs1, dis_r, b1, W2, hs2_o):
    dis = dis_r[...]
    t = jnp.maximum(dis * (a[0] + b[0] + hs1[...]) + b1[...], 0.0)
    hs2_o[...] = jnp.dot(dis * t, W2[...], preferred_element_type=jnp.float32)


def _tc3_body(a, b, hs2, dis_r, b2, Wp1, bp1, Wp2, bp2, out_o):
    dis = dis_r[...]
    o2 = jnp.maximum(dis * (a[0] + b[0] + hs2[...]) + b2[...], 0.0)
    p = jnp.maximum(
        jnp.dot(o2, Wp1[...], preferred_element_type=jnp.float32) + bp1[...],
        0.0)
    out_o[...] = jnp.dot(p, Wp2[...], preferred_element_type=jnp.float32) \
        + bp2[...]


def _row(shape):
    return pl.BlockSpec(shape, lambda i: (i, 0))


def _full(shape):
    return pl.BlockSpec(shape, lambda i: (0, 0))


_GRID = N // BLK


def _plane(d, p):
    return pl.BlockSpec((1, BLK, d), lambda i, _p=p: (_p, i, 0))


_tc1 = pl.pallas_call(
    _tc1_body,
    grid=(_GRID,),
    in_specs=[
        _plane(16, 0),
        _plane(16, 1),
        _row((BLK, 128)),
        _full((128, 128)),
    ],
    out_specs=[_row((BLK, 1)), _row((BLK, 128))],
    out_shape=[
        jax.ShapeDtypeStruct((N, 1), jnp.float32),
        jax.ShapeDtypeStruct((N, 128), jnp.float32),
    ],
)

_tc2 = pl.pallas_call(
    _tc2_body,
    grid=(_GRID,),
    in_specs=[
        _plane(128, 0),
        _plane(128, 1),
        _row((BLK, 128)),
        _row((BLK, 1)),
        _full((1, 128)),
        _full((128, 64)),
    ],
    out_specs=_row((BLK, 64)),
    out_shape=jax.ShapeDtypeStruct((N, 64), jnp.float32),
)

_tc3 = pl.pallas_call(
    _tc3_body,
    grid=(_GRID,),
    in_specs=[
        _plane(64, 0),
        _plane(64, 1),
        _row((BLK, 64)),
        _row((BLK, 1)),
        _full((1, 64)),
        _full((64, 32)),
        _full((1, 32)),
        _full((32, 16)),
        _full((1, 16)),
    ],
    out_specs=_row((BLK, 16)),
    out_shape=jax.ShapeDtypeStruct((N, 16), jnp.float32),
)


def kernel(x, edge_index, W1, b1, W2, b2, Wp1, bp1, Wp2, bp2):
    src3 = edge_index[0].astype(jnp.int32).reshape(NW, CHUNKS, B)
    dst3 = edge_index[1].astype(jnp.int32).reshape(NW, CHUNKS, B)

    ones16 = jnp.stack([jnp.ones((B, 16), jnp.float32),
                        jnp.zeros((B, 16), jnp.float32)])
    z128 = jnp.zeros((CS, 128), jnp.float32)
    z64 = jnp.zeros((CS, 64), jnp.float32)

    degp = _deg_kernel(dst3, ones16)
    dis, hs1 = _tc1(degp, degp, x, W1)
    a1 = _scatter128(hs1, src3, dst3, z128)
    hs2 = _tc2(a1, a1, hs1, dis, b1.reshape(1, -1), W2)
    a2 = _scatter64(hs2, src3, dst3, z64)
    return _tc3(a2, a2, hs2, dis, b2.reshape(1, -1), Wp1,
                bp1.reshape(1, -1), Wp2, bp2.reshape(1, -1))


# speedup vs baseline: 31.3632x; 1.0289x over previous
import jax
import jax.numpy as jnp
from jax import lax
from jax.experimental import pallas as pl
from jax.experimental.pallas import tpu as pltpu
from jax.experimental.pallas import tpu_sc as plsc

N = 10000
E = 320000
NACC = 10240
NC = 2
NS = 16
NW = NC * NS
B = 125
CHUNKS = 80
RPT = NACC // NS
SB = 2
SBC = CHUNKS // SB
NBUF = 2
CS = 80
BLK = 1000

_mesh = plsc.VectorSubcoreMesh(core_axis_name="c", subcore_axis_name="s")


def _deg_body(ei_hbm, ones_hbm, out_hbm, idx_v, ones_v, buf_v, acc_sh,
              gsem, ssem):
    c = lax.axis_index("c")
    s = lax.axis_index("s")
    w = s * NC + c
    dst_hbm = ei_hbm.at[1]
    pltpu.sync_copy(ones_hbm.at[0], ones_v)
    pltpu.sync_copy(ones_hbm.at[1], buf_v)
    base = s * RPT
    zw = []
    for k in range(RPT // CS):
        zw.append(pltpu.async_copy(
            buf_v.at[pl.ds(0, CS)], acc_sh.at[pl.ds(base + k * CS, CS)],
            ssem))
    for d in zw:
        d.wait()
    pltpu.sync_copy(dst_hbm.at[w], idx_v)
    plsc.subcore_barrier()

    def step(p, carry):
        ds = [pltpu.async_copy(
            ones_v, acc_sh.at[idx_v.at[8 * p + i]], ssem, add=True)
            for i in range(8)]
        for d in ds:
            d.wait()
        return carry

    lax.fori_loop(0, CHUNKS // 8, step, 0)
    plsc.subcore_barrier()
    out_pl = out_hbm.at[c]
    rd = {}
    wr = {}
    nck = RPT // CS
    for k in range(nck):
        bv = buf_v if k % 2 == 0 else ones_v
        if k >= 2:
            wr[k - 2].wait()
        rd[k] = pltpu.async_copy(
            acc_sh.at[pl.ds(base + k * CS, CS)], bv.at[pl.ds(0, CS)], gsem)
        rd[k].wait()
        wr[k] = pltpu.async_copy(
            bv.at[pl.ds(0, CS)], out_pl.at[pl.ds(base + k * CS, CS)], ssem)
    wr[nck - 2].wait()
    wr[nck - 1].wait()


_deg_kernel = pl.kernel(
    _deg_body,
    out_type=jax.ShapeDtypeStruct((2, NACC, 16), jnp.float32),
    mesh=_mesh,
    scratch_types=[
        pltpu.VMEM((CHUNKS, B), jnp.int32),
        pltpu.VMEM((B, 16), jnp.float32),
        pltpu.VMEM((B, 16), jnp.float32),
        pltpu.VMEM_SHARED((NACC, 16), jnp.float32),
        pltpu.SemaphoreType.DMA,
        pltpu.SemaphoreType.DMA,
    ],
)


def _make_scatter(D, tc_tiling=True):
    def body(hs_hbm, ei_hbm, zero_hbm, out_hbm,
             sidx_v, didx_v, rows_v, acc_sh, gsem, ssem):
        c = lax.axis_index("c")
        s = lax.axis_index("s")
        w = s * NC + c
        src_hbm = ei_hbm.at[0]
        dst_hbm = ei_hbm.at[1]
        pltpu.sync_copy(zero_hbm, rows_v.at[0].at[pl.ds(0, CS)])
        base = s * RPT
        zw = []
        for k in range(RPT // CS):
            zw.append(pltpu.async_copy(
                rows_v.at[0].at[pl.ds(0, CS)],
                acc_sh.at[pl.ds(base + k * CS, CS)], ssem))
        for d in zw:
            d.wait()
        plsc.subcore_barrier()

        for sb in range(SB):
            pltpu.sync_copy(src_hbm.at[w].at[pl.ds(sb * SBC, SBC)], sidx_v)
            pltpu.sync_copy(dst_hbm.at[w].at[pl.ds(sb * SBC, SBC)], didx_v)

            def group(p, carry):
                j0 = NBUF * p
                gs = [pltpu.async_copy(
                    hs_hbm.at[sidx_v.at[j0 + i]], rows_v.at[i], gsem)
                    for i in range(NBUF)]
                ss = []
                for i in range(NBUF):
                    gs[i].wait()
                    ss.append(pltpu.async_copy(
                        rows_v.at[i], acc_sh.at[didx_v.at[j0 + i]], ssem,
                        add=True))
                for d in ss:
                    d.wait()
                return carry

            lax.fori_loop(0, SBC // NBUF, group, 0)
        plsc.subcore_barrier()

        out_pl = out_hbm.at[c]
        rd = {}
        wr = {}
        nck = RPT // CS
        for k in range(nck):
            bv = rows_v.at[k % NBUF].at[pl.ds(0, CS)]
            if k >= NBUF:
                wr[k - NBUF].wait()
            rd[k] = pltpu.async_copy(
                acc_sh.at[pl.ds(base + k * CS, CS)], bv, gsem)
            rd[k].wait()
            wr[k] = pltpu.async_copy(
                bv, out_pl.at[pl.ds(base + k * CS, CS)], ssem)
        for k in range(max(nck - NBUF, 0), nck):
            wr[k].wait()

    params = None
    if not tc_tiling:
        params = pltpu.CompilerParams(use_tc_tiling_on_sc=False)
    return pl.kernel(
        body,
        out_type=jax.ShapeDtypeStruct((2, NACC, D), jnp.float32),
        mesh=_mesh,
        compiler_params=params,
        scratch_types=[
            pltpu.VMEM((SBC, B), jnp.int32),
            pltpu.VMEM((SBC, B), jnp.int32),
            pltpu.VMEM((NBUF, B, D), jnp.float32),
            pltpu.VMEM_SHARED((NACC, D), jnp.float32),
            pltpu.SemaphoreType.DMA,
            pltpu.SemaphoreType.DMA,
        ],
    )


_scatter128 = _make_scatter(128)
_scatter64 = _make_scatter(64, tc_tiling=False)


def _tc1_body(dega, degb, x, W1, dis_o, hs1_o):
    deg = 1.0 + dega[0, :, 0:1] + degb[0, :, 0:1]
    dis = lax.rsqrt(deg)
    dis_o[...] = dis
    hs1_o[...] = jnp.dot(dis * x[...], W1[...],
                         preferred_element_type=jnp.float32)


def _tc2_body(a, b, hs1, dis_r, b1, W2, hs2_o):
    dis = dis_r[...]
    t = jnp.maximum(dis * (a[0] + b[0] + hs1[...]) + b1[...], 0.0)
    hs2_o[...] = jnp.dot(dis * t, W2[...], preferred_element_type=jnp.float32)


def _tc3_body(a, b, hs2, dis_r, b2, Wp1, bp1, Wp2, bp2, out_o):
    dis = dis_r[...]
    o2 = jnp.maximum(dis * (a[0] + b[0] + hs2[...]) + b2[...], 0.0)
    p = jnp.maximum(
        jnp.dot(o2, Wp1[...], preferred_element_type=jnp.float32) + bp1[...],
        0.0)
    out_o[...] = jnp.dot(p, Wp2[...], preferred_element_type=jnp.float32) \
        + bp2[...]


def _row(shape):
    return pl.BlockSpec(shape, lambda i: (i, 0))


def _full(shape):
    return pl.BlockSpec(shape, lambda i: (0, 0))


_GRID = N // BLK


def _plane(d, p):
    return pl.BlockSpec((1, BLK, d), lambda i, _p=p: (_p, i, 0))


_tc1 = pl.pallas_call(
    _tc1_body,
    grid=(_GRID,),
    in_specs=[
        _plane(16, 0),
        _plane(16, 1),
        _row((BLK, 128)),
        _full((128, 128)),
    ],
    out_specs=[_row((BLK, 1)), _row((BLK, 128))],
    out_shape=[
        jax.ShapeDtypeStruct((N, 1), jnp.float32),
        jax.ShapeDtypeStruct((N, 128), jnp.float32),
    ],
)

_tc2 = pl.pallas_call(
    _tc2_body,
    grid=(_GRID,),
    in_specs=[
        _plane(128, 0),
        _plane(128, 1),
        _row((BLK, 128)),
        _row((BLK, 1)),
        _full((1, 128)),
        _full((128, 64)),
    ],
    out_specs=_row((BLK, 64)),
    out_shape=jax.ShapeDtypeStruct((N, 64), jnp.float32),
)

_tc3 = pl.pallas_call(
    _tc3_body,
    grid=(_GRID,),
    in_specs=[
        _plane(64, 0),
        _plane(64, 1),
        _row((BLK, 64)),
        _row((BLK, 1)),
        _full((1, 64)),
        _full((64, 32)),
        _full((1, 32)),
        _full((32, 16)),
        _full((1, 16)),
    ],
    out_specs=_row((BLK, 16)),
    out_shape=jax.ShapeDtypeStruct((N, 16), jnp.float32),
)


def kernel(x, edge_index, W1, b1, W2, b2, Wp1, bp1, Wp2, bp2):
    ei4 = edge_index.astype(jnp.int32).reshape(2, NW, CHUNKS, B)

    ones16 = jnp.stack([jnp.ones((B, 16), jnp.float32),
                        jnp.zeros((B, 16), jnp.float32)])
    z128 = jnp.zeros((CS, 128), jnp.float32)
    z64 = jnp.zeros((CS, 64), jnp.float32)

    degp = _deg_kernel(ei4, ones16)
    dis, hs1 = _tc1(degp, degp, x, W1)
    a1 = _scatter128(hs1, ei4, z128)
    hs2 = _tc2(a1, a1, hs1, dis, b1.reshape(1, -1), W2)
    a2 = _scatter64(hs2, ei4, z64)
    return _tc3(a2, a2, hs2, dis, b2.reshape(1, -1), Wp1,
                bp1.reshape(1, -1), Wp2, bp2.reshape(1, -1))


# speedup vs baseline: 31.7614x; 1.0127x over previous
import jax
import jax.numpy as jnp
from jax import lax
from jax.experimental import pallas as pl
from jax.experimental.pallas import tpu as pltpu
from jax.experimental.pallas import tpu_sc as plsc

N = 10000
E = 320000
NACC = 10240
NC = 2
NS = 16
NW = NC * NS
B = 125
CHUNKS = 80
RPT = NACC // NS
SB = 2
SBC = CHUNKS // SB
NBUF = 2
CS = 80
BLK = 1000

_mesh = plsc.VectorSubcoreMesh(core_axis_name="c", subcore_axis_name="s")


def _deg_body(ei_hbm, ones_hbm, out_hbm, idx_v, ones_v, buf_v, acc_sh,
              gsem, ssem):
    c = lax.axis_index("c")
    s = lax.axis_index("s")
    w = s * NC + c
    dst_hbm = ei_hbm.at[1]
    pltpu.sync_copy(ones_hbm.at[0], ones_v)
    pltpu.sync_copy(ones_hbm.at[1], buf_v)
    base = s * RPT
    zw = []
    for k in range(RPT // CS):
        zw.append(pltpu.async_copy(
            buf_v.at[pl.ds(0, CS)], acc_sh.at[pl.ds(base + k * CS, CS)],
            ssem))
    for d in zw:
        d.wait()
    pltpu.sync_copy(dst_hbm.at[w], idx_v)
    plsc.subcore_barrier()

    def step(p, carry):
        ds = [pltpu.async_copy(
            ones_v, acc_sh.at[idx_v.at[8 * p + i]], ssem, add=True)
            for i in range(8)]
        for d in ds:
            d.wait()
        return carry

    lax.fori_loop(0, CHUNKS // 8, step, 0)
    plsc.subcore_barrier()
    out_pl = out_hbm.at[c]
    rd = {}
    wr = {}
    nck = RPT // CS
    for k in range(nck):
        bv = buf_v if k % 2 == 0 else ones_v
        if k >= 2:
            wr[k - 2].wait()
        rd[k] = pltpu.async_copy(
            acc_sh.at[pl.ds(base + k * CS, CS)], bv.at[pl.ds(0, CS)], gsem)
        rd[k].wait()
        wr[k] = pltpu.async_copy(
            bv.at[pl.ds(0, CS)], out_pl.at[pl.ds(base + k * CS, CS)], ssem)
    wr[nck - 2].wait()
    wr[nck - 1].wait()


_deg_kernel = pl.kernel(
    _deg_body,
    out_type=jax.ShapeDtypeStruct((2, NACC, 16), jnp.float32),
    mesh=_mesh,
    scratch_types=[
        pltpu.VMEM((CHUNKS, B), jnp.int32),
        pltpu.VMEM((B, 16), jnp.float32),
        pltpu.VMEM((B, 16), jnp.float32),
        pltpu.VMEM_SHARED((NACC, 16), jnp.float32),
        pltpu.SemaphoreType.DMA,
        pltpu.SemaphoreType.DMA,
    ],
)


def _make_scatter(D, tc_tiling=True):
    def body(hs_hbm, ei_hbm, zero_hbm, out_hbm,
             sidx_v, didx_v, rows_v, acc_sh, gsem, ssem):
        c = lax.axis_index("c")
        s = lax.axis_index("s")
        w = s * NC + c
        src_hbm = ei_hbm.at[0]
        dst_hbm = ei_hbm.at[1]
        pltpu.sync_copy(zero_hbm, rows_v.at[0].at[pl.ds(0, CS)])
        base = s * RPT
        zw = []
        for k in range(RPT // CS):
            zw.append(pltpu.async_copy(
                rows_v.at[0].at[pl.ds(0, CS)],
                acc_sh.at[pl.ds(base + k * CS, CS)], ssem))
        pltpu.sync_copy(src_hbm.at[w].at[pl.ds(0, SBC)], sidx_v)
        pltpu.sync_copy(dst_hbm.at[w].at[pl.ds(0, SBC)], didx_v)
        for d in zw:
            d.wait()
        plsc.subcore_barrier()

        for sb in range(SB):
            if sb > 0:
                pltpu.sync_copy(src_hbm.at[w].at[pl.ds(sb * SBC, SBC)],
                                sidx_v)
                pltpu.sync_copy(dst_hbm.at[w].at[pl.ds(sb * SBC, SBC)],
                                didx_v)

            def group(p, carry):
                j0 = NBUF * p
                gs = [pltpu.async_copy(
                    hs_hbm.at[sidx_v.at[j0 + i]], rows_v.at[i], gsem)
                    for i in range(NBUF)]
                ss = []
                for i in range(NBUF):
                    gs[i].wait()
                    ss.append(pltpu.async_copy(
                        rows_v.at[i], acc_sh.at[didx_v.at[j0 + i]], ssem,
                        add=True))
                for d in ss:
                    d.wait()
                return carry

            lax.fori_loop(0, SBC // NBUF, group, 0)
        plsc.subcore_barrier()

        out_pl = out_hbm.at[c]
        rd = {}
        wr = {}
        nck = RPT // CS
        for k in range(nck):
            bv = rows_v.at[k % NBUF].at[pl.ds(0, CS)]
            if k >= NBUF:
                wr[k - NBUF].wait()
            rd[k] = pltpu.async_copy(
                acc_sh.at[pl.ds(base + k * CS, CS)], bv, gsem)
            rd[k].wait()
            wr[k] = pltpu.async_copy(
                bv, out_pl.at[pl.ds(base + k * CS, CS)], ssem)
        for k in range(max(nck - NBUF, 0), nck):
            wr[k].wait()

    params = None
    if not tc_tiling:
        params = pltpu.CompilerParams(use_tc_tiling_on_sc=False)
    return pl.kernel(
        body,
        out_type=jax.ShapeDtypeStruct((2, NACC, D), jnp.float32),
        mesh=_mesh,
        compiler_params=params,
        scratch_types=[
            pltpu.VMEM((SBC, B), jnp.int32),
            pltpu.VMEM((SBC, B), jnp.int32),
            pltpu.VMEM((NBUF, B, D), jnp.float32),
            pltpu.VMEM_SHARED((NACC, D), jnp.float32),
            pltpu.SemaphoreType.DMA,
            pltpu.SemaphoreType.DMA,
        ],
    )


_scatter128 = _make_scatter(128)
_scatter64 = _make_scatter(64, tc_tiling=False)


def _tc1_body(dega, degb, x, W1, dis_o, hs1_o):
    deg = 1.0 + dega[0, :, 0:1] + degb[0, :, 0:1]
    dis = lax.rsqrt(deg)
    dis_o[...] = dis
    hs1_o[...] = jnp.dot(dis * x[...], W1[...],
                         preferred_element_type=jnp.float32)


def _tc2_body(a, b, hs1, dis_r, b1, W2, hs2_o):
    dis = dis_r[...]
    t = jnp.maximum(dis * (a[0] + b[0] + hs1[...]) + b1[...], 0.0)
    hs2_o[...] = jnp.dot(dis * t, W2[...], preferred_element_type=jnp.float32)


def _tc3_body(a, b, hs2, dis_r, b2, Wp1, bp1, Wp2, bp2, out_o):
    dis = dis_r[...]
    o2 = jnp.maximum(dis * (a[0] + b[0] + hs2[...]) + b2[...], 0.0)
    p = jnp.maximum(
        jnp.dot(o2, Wp1[...], preferred_element_type=jnp.float32) + bp1[...],
        0.0)
    out_o[...] = jnp.dot(p, Wp2[...], preferred_element_type=jnp.float32) \
        + bp2[...]


def _row(shape):
    return pl.BlockSpec(shape, lambda i: (i, 0))


def _full(shape):
    return pl.BlockSpec(shape, lambda i: (0, 0))


_GRID = N // BLK


def _plane(d, p):
    return pl.BlockSpec((1, BLK, d), lambda i, _p=p: (_p, i, 0))


_tc1 = pl.pallas_call(
    _tc1_body,
    grid=(_GRID,),
    in_specs=[
        _plane(16, 0),
        _plane(16, 1),
        _row((BLK, 128)),
        _full((128, 128)),
    ],
    out_specs=[_row((BLK, 1)), _row((BLK, 128))],
    out_shape=[
        jax.ShapeDtypeStruct((N, 1), jnp.float32),
        jax.ShapeDtypeStruct((N, 128), jnp.float32),
    ],
)

_tc2 = pl.pallas_call(
    _tc2_body,
    grid=(_GRID,),
    in_specs=[
        _plane(128, 0),
        _plane(128, 1),
        _row((BLK, 128)),
        _row((BLK, 1)),
        _full((1, 128)),
        _full((128, 64)),
    ],
    out_specs=_row((BLK, 64)),
    out_shape=jax.ShapeDtypeStruct((N, 64), jnp.float32),
)

_tc3 = pl.pallas_call(
    _tc3_body,
    grid=(_GRID,),
    in_specs=[
        _plane(64, 0),
        _plane(64, 1),
        _row((BLK, 64)),
        _row((BLK, 1)),
        _full((1, 64)),
        _full((64, 32)),
        _full((1, 32)),
        _full((32, 16)),
        _full((1, 16)),
    ],
    out_specs=_row((BLK, 16)),
    out_shape=jax.ShapeDtypeStruct((N, 16), jnp.float32),
)


def kernel(x, edge_index, W1, b1, W2, b2, Wp1, bp1, Wp2, bp2):
    ei4 = edge_index.astype(jnp.int32).reshape(2, NW, CHUNKS, B)

    ones16 = jnp.stack([jnp.ones((B, 16), jnp.float32),
                        jnp.zeros((B, 16), jnp.float32)])
    z128 = jnp.zeros((CS, 128), jnp.float32)
    z64 = jnp.zeros((CS, 64), jnp.float32)

    degp = _deg_kernel(ei4, ones16)
    dis, hs1 = _tc1(degp, degp, x, W1)
    a1 = _scatter128(hs1, ei4, z128)
    hs2 = _tc2(a1, a1, hs1, dis, b1.reshape(1, -1), W2)
    a2 = _scatter64(hs2, ei4, z64)
    return _tc3(a2, a2, hs2, dis, b2.reshape(1, -1), Wp1,
                bp1.reshape(1, -1), Wp2, bp2.reshape(1, -1))
